# baseline (speedup 1.0000x reference)
import functools

import jax
import jax.numpy as jnp
from jax import lax
from jax.experimental import pallas as pl
from jax.experimental.pallas import tpu as pltpu
from jax.experimental.pallas import tpu_sc as plsc

KEEP_THRESHOLD = 0.01
NEG = -10000.0
L = 16
NSUB = 16
R = 1792


def _tc_body(mask_ref, probs_ref, out_ref, acc_ref):
    c = pl.program_id(1)

    tile = probs_ref[0, 0, :, :]
    m = mask_ref[0, :, :]
    masked = jnp.where(m < 0.0, 0.0, tile)
    partial = jnp.sum(masked, axis=0, keepdims=True)

    @pl.when(c == 0)
    def _init():
        acc_ref[...] = partial

    @pl.when(c != 0)
    def _accum():
        acc_ref[...] += partial

    @pl.when(c == pl.num_programs(1) - 1)
    def _epilogue():
        out_ref[0, :, :] = acc_ref[...]


def _sc_body(mask_hbm, table_hbm, out_hbm,
             mask_v, buf_v, acc_v, shared, *, H, S):
    b = lax.axis_index("c")
    sid = lax.axis_index("s")
    nrows = (S - R) // NSUB
    r0 = R + sid * nrows
    ncol = S // L
    zf = jnp.zeros((L,), jnp.float32)
    onef = jnp.ones((L,), jnp.float32)

    def zero(j, _):
        acc_v[pl.ds(j * L, L)] = zf
        return 0
    lax.fori_loop(0, ncol, zero, 0)

    pltpu.sync_copy(mask_hbm.at[b, pl.ds(r0, L)], mask_v)
    mvv = mask_v[...]
    ks = [zf + jnp.where(mvv[i] >= 0.0, 1.0, 0.0)
          for i in range(nrows)]

    def head(h, _):
        row = (b * H + h) * S + r0
        pltpu.sync_copy(table_hbm.at[pl.ds(row, nrows)], buf_v)

        def col(j, _2):
            base = j * L
            reg = acc_v[pl.ds(base, L)]
            for i in range(nrows):
                reg = reg + buf_v[i, pl.ds(base, L)] * ks[i]
            acc_v[pl.ds(base, L)] = reg
            return 0
        lax.fori_loop(0, ncol, col, 0)
        return 0
    lax.fori_loop(0, H, head, 0)

    pltpu.sync_copy(acc_v, shared.at[sid])
    plsc.subcore_barrier()

    @pl.when(sid == 0)
    def _final():
        pltpu.sync_copy(shared, buf_v)

        def col2(j, _2):
            base = j * L
            reg = buf_v[0, pl.ds(base, L)]
            for rr in range(1, NSUB):
                reg = reg + buf_v[rr, pl.ds(base, L)]
            acc_v[pl.ds(base, L)] = reg
            return 0
        lax.fori_loop(0, ncol, col2, 0)
        pltpu.sync_copy(acc_v, out_hbm.at[b])


def _combine_body(tc_ref, sc_ref, out_ref):
    scores = tc_ref[...] + sc_ref[...]
    mx = jnp.max(scores, axis=-1, keepdims=True)
    rel = scores / mx
    out_ref[...] = jnp.where(rel < KEEP_THRESHOLD, NEG, 0.0)


def kernel(attention_mask, attention_probs, sentence_lengths):
    del sentence_lengths
    B, H, S, _ = attention_probs.shape

    mask3 = attention_mask.reshape(B, S, 1)
    mask2 = attention_mask.reshape(B, S)
    table = attention_probs.reshape(B * H * S, S)

    tc_part = pl.pallas_call(
        _tc_body,
        grid=(B, H),
        in_specs=[
            pl.BlockSpec((1, R, 1), lambda b, c: (b, 0, 0)),
            pl.BlockSpec((1, 1, R, S), lambda b, c: (b, c, 0, 0)),
        ],
        out_specs=pl.BlockSpec((1, 1, S), lambda b, c: (b, 0, 0)),
        out_shape=jax.ShapeDtypeStruct((B, 1, S), jnp.float32),
        scratch_shapes=[pltpu.VMEM((1, S), jnp.float32)],
        compiler_params=pltpu.CompilerParams(
            dimension_semantics=("parallel", "arbitrary")),
    )(mask3, attention_probs)

    mesh = plsc.VectorSubcoreMesh(
        core_axis_name="c", subcore_axis_name="s",
        num_cores=2, num_subcores=NSUB)
    sc_part = pl.kernel(
        functools.partial(_sc_body, H=H, S=S),
        out_type=jax.ShapeDtypeStruct((B, S), jnp.float32),
        mesh=mesh,
        scratch_types=[
            pltpu.VMEM((L,), jnp.float32),
            pltpu.VMEM((NSUB, S), jnp.float32),
            pltpu.VMEM((S,), jnp.float32),
            pltpu.VMEM_SHARED((NSUB, S), jnp.float32),
        ],
    )(mask2, table)

    out = pl.pallas_call(
        _combine_body,
        out_shape=jax.ShapeDtypeStruct((B, S), jnp.float32),
    )(tc_part.reshape(B, S), sc_part)
    return out.reshape(B, 1, 1, S)


# speedup vs baseline: 1.0032x; 1.0032x over previous
import functools

import jax
import jax.numpy as jnp
from jax import lax
from jax.experimental import pallas as pl
from jax.experimental.pallas import tpu as pltpu
from jax.experimental.pallas import tpu_sc as plsc

KEEP_THRESHOLD = 0.01
NEG = -10000.0
L = 16
NSUB = 16
R = 1792


def _tc_body(mask_ref, probs_ref, out_ref, acc_ref):
    c = pl.program_id(1)

    tile = probs_ref[0, 0, :, :]
    m = mask_ref[0, :, :]
    masked = jnp.where(m < 0.0, 0.0, tile)
    partial = jnp.sum(masked, axis=0, keepdims=True)

    @pl.when(c == 0)
    def _init():
        acc_ref[...] = partial

    @pl.when(c != 0)
    def _accum():
        acc_ref[...] += partial

    @pl.when(c == pl.num_programs(1) - 1)
    def _epilogue():
        out_ref[0, :, :] = acc_ref[...]


def _sc_body(mask_hbm, table_hbm, out_hbm,
             mask_v, buf_v, acc_v, shared, *, H, S):
    b = lax.axis_index("c")
    sid = lax.axis_index("s")
    nrows = (S - R) // NSUB
    r0 = R + sid * nrows
    ncol = S // L
    zf = jnp.zeros((L,), jnp.float32)
    onef = jnp.ones((L,), jnp.float32)

    def zero(j, _):
        acc_v[pl.ds(j * L, L)] = zf
        return 0
    lax.fori_loop(0, ncol, zero, 0)

    pltpu.sync_copy(mask_hbm.at[b, pl.ds(r0, L)], mask_v)
    mvv = mask_v[...]
    ks = [zf + jnp.where(mvv[i] >= 0.0, 1.0, 0.0)
          for i in range(nrows)]

    def head(h, _):
        row = (b * H + h) * S + r0
        pltpu.sync_copy(table_hbm.at[pl.ds(row, nrows)], buf_v)

        def col(j, _2):
            base = j * L
            reg = acc_v[pl.ds(base, L)]
            for i in range(nrows):
                reg = reg + buf_v[i, pl.ds(base, L)] * ks[i]
            acc_v[pl.ds(base, L)] = reg
            return 0
        lax.fori_loop(0, ncol, col, 0)
        return 0
    lax.fori_loop(0, H, head, 0)

    pltpu.sync_copy(acc_v, shared.at[sid])
    plsc.subcore_barrier()

    @pl.when(sid == 0)
    def _final():
        pltpu.sync_copy(shared, buf_v)

        def col2(j, _2):
            base = j * L
            reg = buf_v[0, pl.ds(base, L)]
            for rr in range(1, NSUB):
                reg = reg + buf_v[rr, pl.ds(base, L)]
            acc_v[pl.ds(base, L)] = reg
            return 0
        lax.fori_loop(0, ncol, col2, 0)
        pltpu.sync_copy(acc_v, out_hbm.at[b])


def _combine_body(tc_ref, sc_ref, out_ref):
    scores = tc_ref[...] + sc_ref[...]
    mx = jnp.max(scores, axis=-1, keepdims=True)
    rel = scores / mx
    out_ref[...] = jnp.where(rel < KEEP_THRESHOLD, NEG, 0.0)


def kernel(attention_mask, attention_probs, sentence_lengths):
    del sentence_lengths
    B, H, S, _ = attention_probs.shape

    mask3 = attention_mask.reshape(B, S, 1)
    mask2 = attention_mask.reshape(B, S)
    table = attention_probs.reshape(B * H * S, S)

    mesh = plsc.VectorSubcoreMesh(
        core_axis_name="c", subcore_axis_name="s",
        num_cores=2, num_subcores=NSUB)
    sc_part = pl.kernel(
        functools.partial(_sc_body, H=H, S=S),
        out_type=jax.ShapeDtypeStruct((B, S), jnp.float32),
        mesh=mesh,
        scratch_types=[
            pltpu.VMEM((L,), jnp.float32),
            pltpu.VMEM((NSUB, S), jnp.float32),
            pltpu.VMEM((S,), jnp.float32),
            pltpu.VMEM_SHARED((NSUB, S), jnp.float32),
        ],
    )(mask2, table)

    tc_part = pl.pallas_call(
        _tc_body,
        grid=(B, H),
        in_specs=[
            pl.BlockSpec((1, R, 1), lambda b, c: (b, 0, 0)),
            pl.BlockSpec((1, 1, R, S), lambda b, c: (b, c, 0, 0)),
        ],
        out_specs=pl.BlockSpec((1, 1, S), lambda b, c: (b, 0, 0)),
        out_shape=jax.ShapeDtypeStruct((B, 1, S), jnp.float32),
        scratch_shapes=[pltpu.VMEM((1, S), jnp.float32)],
        compiler_params=pltpu.CompilerParams(
            dimension_semantics=("parallel", "arbitrary")),
    )(mask3, attention_probs)

    out = pl.pallas_call(
        _combine_body,
        out_shape=jax.ShapeDtypeStruct((B, S), jnp.float32),
    )(tc_part.reshape(B, S), sc_part)
    return out.reshape(B, 1, 1, S)


# speedup vs baseline: 1.0629x; 1.0595x over previous
import jax
import jax.numpy as jnp
from jax.experimental import pallas as pl
from jax.experimental.pallas import tpu as pltpu

KEEP_THRESHOLD = 0.01
NEG = -10000.0
RB = 512


def _body(mask_ref, probs_ref, out_ref, acc_ref):
    h = pl.program_id(1)
    r = pl.program_id(2)

    tile = probs_ref[0, 0, :, :]
    m = mask_ref[0, :, :]
    k = jnp.where(m >= 0.0, 1.0, 0.0)
    partial = jnp.dot(k, tile,
                      preferred_element_type=jnp.float32)

    first = jnp.logical_and(h == 0, r == 0)
    @pl.when(first)
    def _init():
        acc_ref[...] = partial

    @pl.when(jnp.logical_not(first))
    def _accum():
        acc_ref[...] += partial

    last = jnp.logical_and(h == pl.num_programs(1) - 1,
                           r == pl.num_programs(2) - 1)
    @pl.when(last)
    def _epilogue():
        scores = acc_ref[...]
        mx = jnp.max(scores)
        out_ref[0, :, :] = jnp.where(scores < KEEP_THRESHOLD * mx,
                                     NEG, 0.0)


def kernel(attention_mask, attention_probs, sentence_lengths):
    del sentence_lengths
    B, H, S, _ = attention_probs.shape
    mask2 = attention_mask.reshape(B, 1, S)

    out = pl.pallas_call(
        _body,
        grid=(B, H, S // RB),
        in_specs=[
            pl.BlockSpec((1, 1, RB), lambda b, h, r: (b, 0, r)),
            pl.BlockSpec((1, 1, RB, S), lambda b, h, r: (b, h, r, 0)),
        ],
        out_specs=pl.BlockSpec((1, 1, S), lambda b, h, r: (b, 0, 0)),
        out_shape=jax.ShapeDtypeStruct((B, 1, S), jnp.float32),
        scratch_shapes=[pltpu.VMEM((1, S), jnp.float32)],
        compiler_params=pltpu.CompilerParams(
            dimension_semantics=("parallel", "arbitrary", "arbitrary")),
    )(mask2, attention_probs)
    return out.reshape(B, 1, 1, S)


# speedup vs baseline: 1.2046x; 1.1334x over previous
import jax
import jax.numpy as jnp
from jax.experimental import pallas as pl
from jax.experimental.pallas import tpu as pltpu

KEEP_THRESHOLD = 0.01
NEG = -10000.0
RB = 2048


def _body(mask_ref, probs_ref, out_ref, acc_ref):
    h = pl.program_id(1)
    r = pl.program_id(2)

    tile = probs_ref[0, 0, :, :]
    m = mask_ref[0, :, :]
    k = jnp.where(m >= 0.0, 1.0, 0.0)
    partial = jnp.dot(k, tile,
                      preferred_element_type=jnp.float32)

    first = jnp.logical_and(h == 0, r == 0)
    @pl.when(first)
    def _init():
        acc_ref[...] = partial

    @pl.when(jnp.logical_not(first))
    def _accum():
        acc_ref[...] += partial

    last = jnp.logical_and(h == pl.num_programs(1) - 1,
                           r == pl.num_programs(2) - 1)
    @pl.when(last)
    def _epilogue():
        scores = acc_ref[...]
        mx = jnp.max(scores)
        out_ref[0, :, :] = jnp.where(scores < KEEP_THRESHOLD * mx,
                                     NEG, 0.0)


def kernel(attention_mask, attention_probs, sentence_lengths):
    del sentence_lengths
    B, H, S, _ = attention_probs.shape
    mask2 = attention_mask.reshape(B, 1, S)

    out = pl.pallas_call(
        _body,
        grid=(B, H, S // RB),
        in_specs=[
            pl.BlockSpec((1, 1, RB), lambda b, h, r: (b, 0, r)),
            pl.BlockSpec((1, 1, RB, S), lambda b, h, r: (b, h, r, 0)),
        ],
        out_specs=pl.BlockSpec((1, 1, S), lambda b, h, r: (b, 0, 0)),
        out_shape=jax.ShapeDtypeStruct((B, 1, S), jnp.float32),
        scratch_shapes=[pltpu.VMEM((1, S), jnp.float32)],
        compiler_params=pltpu.CompilerParams(
            dimension_semantics=("parallel", "arbitrary", "arbitrary")),
    )(mask2, attention_probs)
    return out.reshape(B, 1, 1, S)


# speedup vs baseline: 1.2126x; 1.0067x over previous
import jax
import jax.numpy as jnp
from jax.experimental import pallas as pl
from jax.experimental.pallas import tpu as pltpu

KEEP_THRESHOLD = 0.01
NEG = -10000.0
RB = 1024


def _body(mask_ref, probs_ref, out_ref, acc_ref):
    h = pl.program_id(1)
    r = pl.program_id(2)

    tile = probs_ref[0, 0, :, :]
    m = mask_ref[0, :, :]
    k = jnp.where(m >= 0.0, 1.0, 0.0)
    partial = jnp.dot(k, tile,
                      preferred_element_type=jnp.float32)

    first = jnp.logical_and(h == 0, r == 0)
    @pl.when(first)
    def _init():
        acc_ref[...] = partial

    @pl.when(jnp.logical_not(first))
    def _accum():
        acc_ref[...] += partial

    last = jnp.logical_and(h == pl.num_programs(1) - 1,
                           r == pl.num_programs(2) - 1)
    @pl.when(last)
    def _epilogue():
        scores = acc_ref[...]
        mx = jnp.max(scores)
        out_ref[0, :, :] = jnp.where(scores < KEEP_THRESHOLD * mx,
                                     NEG, 0.0)


def kernel(attention_mask, attention_probs, sentence_lengths):
    del sentence_lengths
    B, H, S, _ = attention_probs.shape
    mask2 = attention_mask.reshape(B, 1, S)

    out = pl.pallas_call(
        _body,
        grid=(B, H, S // RB),
        in_specs=[
            pl.BlockSpec((1, 1, RB), lambda b, h, r: (b, 0, r)),
            pl.BlockSpec((1, 1, RB, S), lambda b, h, r: (b, h, r, 0)),
        ],
        out_specs=pl.BlockSpec((1, 1, S), lambda b, h, r: (b, 0, 0)),
        out_shape=jax.ShapeDtypeStruct((B, 1, S), jnp.float32),
        scratch_shapes=[pltpu.VMEM((1, S), jnp.float32)],
        compiler_params=pltpu.CompilerParams(
            dimension_semantics=("parallel", "arbitrary", "arbitrary")),
    )(mask2, attention_probs)
    return out.reshape(B, 1, 1, S)
